# Initial kernel scaffold; baseline (speedup 1.0000x reference)
import jax
import jax.numpy as jnp
from jax.experimental import pallas as pl


def kernel(feat, adj, edges, enc_w1, enc_b1, enc_w2, enc_b2, reg_w, reg_b, g_w1, g_b1, g_w2, g_b2, g_w3, g_b3, cls_w1, cls_b1, cls_w2, cls_b2):
    raise NotImplementedError("write your pallas kernel here")


# speedup vs baseline: 105.5121x; 105.5121x over previous
import functools

import jax
import jax.numpy as jnp
import numpy as np
from jax import lax
from jax.experimental import pallas as pl
from jax.experimental.pallas import tpu as pltpu

_N = 10000
_D = 128
_HID = 64
_LAT = 64
_NPRED = 5
_NCLASS = 40
_E = 320000
_NODE2 = _N * (1 + _NPRED)


def _mm_kern(x_ref, w_ref, b_ref, o_ref, *, act):
    r = jnp.dot(x_ref[...], w_ref[...],
                preferred_element_type=jnp.float32) + b_ref[...]
    if act == "relu":
        r = jnp.maximum(r, 0.0)
    elif act == "tanh":
        r = jnp.tanh(r)
    o_ref[...] = r


def _matmul(x, w, b=None, act=None, bm=500):
    m, k = x.shape
    _, n = w.shape
    assert m % bm == 0, (m, bm)
    if b is None:
        b = jnp.zeros((n,), jnp.float32)
    return pl.pallas_call(
        functools.partial(_mm_kern, act=act),
        grid=(m // bm,),
        in_specs=[
            pl.BlockSpec((bm, k), lambda i: (i, 0)),
            pl.BlockSpec((k, n), lambda i: (0, 0)),
            pl.BlockSpec((1, n), lambda i: (0, 0)),
        ],
        out_specs=pl.BlockSpec((bm, n), lambda i: (i, 0)),
        out_shape=jax.ShapeDtypeStruct((m, n), jnp.float32),
        compiler_params=pltpu.CompilerParams(
            dimension_semantics=("parallel",)),
    )(x, w, b.reshape(1, n))


def _gen_kern(z_ref, nz_ref, w1_ref, b1_ref, w2_ref, b2_ref, w3_ref, b3_ref,
              rw_ref, rb_ref, gen_ref, deg_ref, g1_ref, g2_ref):
    z = z_ref[...]
    deg_ref[...] = jnp.maximum(
        jnp.dot(z, rw_ref[...], preferred_element_type=jnp.float32)
        + rb_ref[0, 0], 0.0)
    zn = z + nz_ref[...]
    g1_ref[...] = jnp.maximum(
        jnp.dot(zn, w1_ref[...], preferred_element_type=jnp.float32)
        + b1_ref[...], 0.0)
    g2_ref[...] = jnp.maximum(
        jnp.dot(g1_ref[...], w2_ref[...], preferred_element_type=jnp.float32)
        + b2_ref[...], 0.0)
    gen_ref[...] = jnp.tanh(
        jnp.dot(g2_ref[...], w3_ref[...], preferred_element_type=jnp.float32)
        + b3_ref[...])


def _generator(z, noise, g_w1, g_b1, g_w2, g_b2, g_w3, g_b3, reg_w, reg_b):
    bm = 400
    nout = _NPRED * _D
    return pl.pallas_call(
        _gen_kern,
        grid=(_N // bm,),
        in_specs=[
            pl.BlockSpec((bm, _LAT), lambda i: (i, 0)),
            pl.BlockSpec((bm, _LAT), lambda i: (i, 0)),
            pl.BlockSpec((_LAT, 256), lambda i: (0, 0)),
            pl.BlockSpec((1, 256), lambda i: (0, 0)),
            pl.BlockSpec((256, 2048), lambda i: (0, 0)),
            pl.BlockSpec((1, 2048), lambda i: (0, 0)),
            pl.BlockSpec((2048, nout), lambda i: (0, 0)),
            pl.BlockSpec((1, nout), lambda i: (0, 0)),
            pl.BlockSpec((_LAT, 1), lambda i: (0, 0)),
            pl.BlockSpec((1, 1), lambda i: (0, 0)),
        ],
        out_specs=[
            pl.BlockSpec((bm, nout), lambda i: (i, 0)),
            pl.BlockSpec((bm, 1), lambda i: (i, 0)),
        ],
        out_shape=[
            jax.ShapeDtypeStruct((_N, nout), jnp.float32),
            jax.ShapeDtypeStruct((_N, 1), jnp.float32),
        ],
        scratch_shapes=[pltpu.VMEM((bm, 256), jnp.float32),
                        pltpu.VMEM((bm, 2048), jnp.float32)],
        compiler_params=pltpu.CompilerParams(
            dimension_semantics=("parallel",)),
    )(z, noise, g_w1, g_b1.reshape(1, 256), g_w2, g_b2.reshape(1, 2048),
      g_w3, g_b3.reshape(1, nout), reg_w, reg_b.reshape(1, 1))


def kernel(feat, adj, edges, enc_w1, enc_b1, enc_w2, enc_b2, reg_w, reg_b,
           g_w1, g_b1, g_w2, g_b2, g_w3, g_b3, cls_w1, cls_b1, cls_w2,
           cls_b2):
    t1 = _matmul(feat, enc_w1, bm=1000)
    h = _matmul(adj, t1, enc_b1, act="relu", bm=200)
    t2 = _matmul(h, enc_w2, bm=1000)
    z = _matmul(adj, t2, enc_b2, act="relu", bm=200)

    noise = jax.random.normal(jax.random.key(42), z.shape, dtype=jnp.float32)
    gen_feat, degree = _generator(z, noise, g_w1, g_b1, g_w2, g_b2, g_w3,
                                  g_b3, reg_w, reg_b)

    a = edges[:, 0].astype(jnp.int32)
    bcol = edges[:, 1].astype(jnp.int32)
    kf = a * _N + bcol
    kb = bcol * _N + a
    sf = jnp.sort(kf)
    cf = (jnp.searchsorted(sf, kf, side="right")
          - jnp.searchsorted(sf, kf, side="left"))
    cb = (jnp.searchsorted(sf, kb, side="right")
          - jnp.searchsorted(sf, kb, side="left"))
    w_e = jnp.maximum(cf, cb).astype(jnp.float32) / (cf + cb).astype(
        jnp.float32)

    deg_i = jnp.clip(degree.astype(jnp.int32).reshape(-1), 0, _NPRED)
    jj = jnp.arange(_NPRED, dtype=jnp.int32)
    mask = (jj[None, :] < deg_i[:, None]).astype(jnp.float32)

    x1a = _matmul(feat, cls_w1, bm=1000)
    x1b = _matmul(gen_feat.reshape(-1, _D), cls_w1, bm=1000)

    r_list = jnp.concatenate([a, bcol])
    c_list = jnp.concatenate([bcol, a])
    w_list = jnp.concatenate([w_e, w_e])

    acc1 = jax.ops.segment_sum(w_list[:, None] * x1a[c_list], r_list,
                               num_segments=_N)
    rowsum = (jax.ops.segment_sum(w_list, r_list, num_segments=_N)
              + deg_i.astype(jnp.float32) + 1.0)
    rinv = 1.0 / rowsum

    m1 = jnp.sum(mask[:, :, None] * x1b.reshape(_N, _NPRED, _HID), axis=1)
    h2a = jnp.maximum((acc1 + m1 + x1a) * rinv[:, None] + cls_b1, 0.0)
    minv = 1.0 / (1.0 + mask)
    h2b = jnp.maximum(
        (x1b.reshape(_N, _NPRED, _HID)
         + mask[:, :, None] * x1a[:, None, :]) * minv[:, :, None]
        + cls_b1, 0.0)

    y2a = _matmul(h2a, cls_w2, bm=1000)
    y2b = _matmul(h2b.reshape(-1, _HID), cls_w2, bm=1000)

    acc2 = jax.ops.segment_sum(w_list[:, None] * y2a[c_list], r_list,
                               num_segments=_N)
    m2 = jnp.sum(mask[:, :, None] * y2b.reshape(_N, _NPRED, _NCLASS), axis=1)
    outa = jnp.maximum((acc2 + m2 + y2a) * rinv[:, None] + cls_b2, 0.0)
    outb = jnp.maximum(
        (y2b.reshape(_N, _NPRED, _NCLASS)
         + mask[:, :, None] * y2a[:, None, :]) * minv[:, :, None]
        + cls_b2, 0.0)
    nc_pred = jnp.concatenate([outa, outb.reshape(-1, _NCLASS)], axis=0)

    return (degree, gen_feat, nc_pred)


# speedup vs baseline: 479.1998x; 4.5417x over previous
import functools

import jax
import jax.numpy as jnp
import numpy as np
from jax import lax
from jax.experimental import pallas as pl
from jax.experimental.pallas import tpu as pltpu
from jax.experimental.pallas import tpu_sc as plsc

_N = 10000
_D = 128
_HID = 64
_LAT = 64
_NPRED = 5
_NCLASS = 40
_E = 320000
_NODE2 = _N * (1 + _NPRED)


def _mm_kern(x_ref, w_ref, b_ref, o_ref, *, act):
    r = jnp.dot(x_ref[...], w_ref[...],
                preferred_element_type=jnp.float32) + b_ref[...]
    if act == "relu":
        r = jnp.maximum(r, 0.0)
    elif act == "tanh":
        r = jnp.tanh(r)
    o_ref[...] = r


def _matmul(x, w, b=None, act=None, bm=500):
    m, k = x.shape
    _, n = w.shape
    assert m % bm == 0, (m, bm)
    if b is None:
        b = jnp.zeros((n,), jnp.float32)
    return pl.pallas_call(
        functools.partial(_mm_kern, act=act),
        grid=(m // bm,),
        in_specs=[
            pl.BlockSpec((bm, k), lambda i: (i, 0)),
            pl.BlockSpec((k, n), lambda i: (0, 0)),
            pl.BlockSpec((1, n), lambda i: (0, 0)),
        ],
        out_specs=pl.BlockSpec((bm, n), lambda i: (i, 0)),
        out_shape=jax.ShapeDtypeStruct((m, n), jnp.float32),
        compiler_params=pltpu.CompilerParams(
            dimension_semantics=("parallel",)),
    )(x, w, b.reshape(1, n))


def _gen_kern(z_ref, nz_ref, w1_ref, b1_ref, w2_ref, b2_ref, w3_ref, b3_ref,
              rw_ref, rb_ref, gen_ref, deg_ref, g1_ref, g2_ref):
    z = z_ref[...]
    deg_ref[...] = jnp.maximum(
        jnp.dot(z, rw_ref[...], preferred_element_type=jnp.float32)
        + rb_ref[0, 0], 0.0)
    zn = z + nz_ref[...]
    g1_ref[...] = jnp.maximum(
        jnp.dot(zn, w1_ref[...], preferred_element_type=jnp.float32)
        + b1_ref[...], 0.0)
    g2_ref[...] = jnp.maximum(
        jnp.dot(g1_ref[...], w2_ref[...], preferred_element_type=jnp.float32)
        + b2_ref[...], 0.0)
    gen_ref[...] = jnp.tanh(
        jnp.dot(g2_ref[...], w3_ref[...], preferred_element_type=jnp.float32)
        + b3_ref[...])


def _generator(z, noise, g_w1, g_b1, g_w2, g_b2, g_w3, g_b3, reg_w, reg_b):
    bm = 400
    nout = _NPRED * _D
    return pl.pallas_call(
        _gen_kern,
        grid=(_N // bm,),
        in_specs=[
            pl.BlockSpec((bm, _LAT), lambda i: (i, 0)),
            pl.BlockSpec((bm, _LAT), lambda i: (i, 0)),
            pl.BlockSpec((_LAT, 256), lambda i: (0, 0)),
            pl.BlockSpec((1, 256), lambda i: (0, 0)),
            pl.BlockSpec((256, 2048), lambda i: (0, 0)),
            pl.BlockSpec((1, 2048), lambda i: (0, 0)),
            pl.BlockSpec((2048, nout), lambda i: (0, 0)),
            pl.BlockSpec((1, nout), lambda i: (0, 0)),
            pl.BlockSpec((_LAT, 1), lambda i: (0, 0)),
            pl.BlockSpec((1, 1), lambda i: (0, 0)),
        ],
        out_specs=[
            pl.BlockSpec((bm, nout), lambda i: (i, 0)),
            pl.BlockSpec((bm, 1), lambda i: (i, 0)),
        ],
        out_shape=[
            jax.ShapeDtypeStruct((_N, nout), jnp.float32),
            jax.ShapeDtypeStruct((_N, 1), jnp.float32),
        ],
        scratch_shapes=[pltpu.VMEM((bm, 256), jnp.float32),
                        pltpu.VMEM((bm, 2048), jnp.float32)],
        compiler_params=pltpu.CompilerParams(
            dimension_semantics=("parallel",)),
    )(z, noise, g_w1, g_b1.reshape(1, 256), g_w2, g_b2.reshape(1, 2048),
      g_w3, g_b3.reshape(1, nout), reg_w, reg_b.reshape(1, 1))


_NCHUNK = 128
_PAD_E2 = 643072
_NRACC = 10112


def _sc_spmm(r_idx, c_idx, w_vals, xg):
    width = xg.shape[1]
    nc, ns = 2, 16
    nw = nc * ns
    per_w = _PAD_E2 // nw
    nchunks = per_w // _NCHUNK
    rpt = _NRACC // ns

    def body(r_hbm, c_hbm, w_hbm, fl_hbm, xg_hbm, zer_hbm, out_hbm,
             acc_sh, ri_v, ci_v, w_v, fl_v, g_v, sem):
        cid = lax.axis_index("c")
        sid = lax.axis_index("s")
        wid = sid * nc + cid
        pltpu.sync_copy(zer_hbm, acc_sh.at[pl.ds(sid * rpt, rpt)])
        pltpu.sync_copy(fl_hbm.at[pl.ds(wid * 176, 176)], fl_v)
        plsc.subcore_barrier()
        base = wid * per_w

        def chunk(g, carry):
            off = base + g * _NCHUNK
            pltpu.sync_copy(r_hbm.at[pl.ds(off, _NCHUNK)], ri_v)
            pltpu.sync_copy(c_hbm.at[pl.ds(off, _NCHUNK)], ci_v)
            pltpu.sync_copy(w_hbm.at[pl.ds(off, _NCHUNK)],
                            w_v.at[pl.ds(0, _NCHUNK)])
            pltpu.async_copy(xg_hbm.at[ci_v], g_v, sem).wait()

            flag = fl_v[pl.ds(g, 16)][0]

            @pl.when(flag != 0)
            def _():
                def rowbody(k, c2):
                    wk = w_v[pl.ds(k, 16)][0]
                    for j in range(width // 16):
                        sl = g_v[k, pl.ds(j * 16, 16)]
                        g_v[k, pl.ds(j * 16, 16)] = sl * wk
                    return c2

                lax.fori_loop(0, _NCHUNK, rowbody, 0)

            pltpu.sync_copy(g_v, acc_sh.at[ri_v], add=True)
            return carry

        lax.fori_loop(0, nchunks, chunk, 0)
        plsc.subcore_barrier()
        pltpu.sync_copy(acc_sh.at[pl.ds(sid * rpt, rpt)],
                        out_hbm.at[cid, pl.ds(sid * rpt, rpt)])

    f = pl.kernel(
        body,
        out_type=jax.ShapeDtypeStruct((nc, _NRACC, width), jnp.float32),
        mesh=plsc.VectorSubcoreMesh(core_axis_name="c", subcore_axis_name="s",
                                    num_cores=nc, num_subcores=ns),
        scratch_types=[
            pltpu.VMEM_SHARED((_NRACC, width), jnp.float32),
            pltpu.VMEM((_NCHUNK,), jnp.int32),
            pltpu.VMEM((_NCHUNK,), jnp.int32),
            pltpu.VMEM((_NCHUNK + 16,), jnp.float32),
            pltpu.VMEM((176,), jnp.int32),
            pltpu.VMEM((_NCHUNK, width), jnp.float32),
            pltpu.SemaphoreType.DMA,
        ],
    )
    flags = (w_vals.reshape(-1, _NCHUNK) != 1.0).any(axis=1).astype(jnp.int32)
    flags = flags.reshape(nw, nchunks)
    flags = jnp.concatenate(
        [flags, jnp.zeros((nw, 176 - nchunks), jnp.int32)], axis=1).reshape(-1)
    zer = jnp.zeros((rpt, width), jnp.float32)
    out = f(r_idx, c_idx, w_vals, flags, xg, zer)
    return out[0] + out[1]


def _edge_entries(a, bcol):
    kf = (a * _N + bcol) * 2 + 1
    kb = (bcol * _N + a) * 2
    sk = jnp.sort(jnp.concatenate([kf, kb]))
    q = sk >> 1
    tag = (sk & 1).astype(jnp.int32)
    one = jnp.ones_like(tag)
    segf = jnp.concatenate([jnp.ones((1,), jnp.int32),
                            (q[1:] != q[:-1]).astype(jnp.int32)])
    segb = jnp.concatenate([(q[:-1] != q[1:]).astype(jnp.int32),
                            jnp.ones((1,), jnp.int32)])

    def segop(x, y):
        fx, tx, nx = x
        fy, ty, ny = y
        keep = fy == 1
        return (jnp.bitwise_or(fx, fy),
                jnp.where(keep, ty, tx + ty),
                jnp.where(keep, ny, nx + ny))

    _, tf, nf = lax.associative_scan(segop, (segf, tag, one))
    _, tb, nb = lax.associative_scan(segop,
                                     (segb[::-1], tag[::-1], one[::-1]))
    tb = tb[::-1]
    nb = nb[::-1]
    fcnt = tf + tb - tag
    tot = nf + nb - 1
    bcnt = tot - fcnt
    w = jnp.maximum(fcnt, bcnt).astype(jnp.float32) / tot.astype(jnp.float32)
    r = q // _N
    c = q % _N
    pad = _PAD_E2 - 2 * _E
    r = jnp.concatenate([r, jnp.full((pad,), _N, jnp.int32)])
    c = jnp.concatenate([c, jnp.full((pad,), _N, jnp.int32)])
    w = jnp.concatenate([w, jnp.zeros((pad,), jnp.float32)])
    return r, c, w


def kernel(feat, adj, edges, enc_w1, enc_b1, enc_w2, enc_b2, reg_w, reg_b,
           g_w1, g_b1, g_w2, g_b2, g_w3, g_b3, cls_w1, cls_b1, cls_w2,
           cls_b2):
    t1 = _matmul(feat, enc_w1, bm=1000)
    h = _matmul(adj, t1, enc_b1, act="relu", bm=200)
    t2 = _matmul(h, enc_w2, bm=1000)
    z = _matmul(adj, t2, enc_b2, act="relu", bm=200)

    noise = jax.random.normal(jax.random.key(42), z.shape, dtype=jnp.float32)
    gen_feat, degree = _generator(z, noise, g_w1, g_b1, g_w2, g_b2, g_w3,
                                  g_b3, reg_w, reg_b)

    a = edges[:, 0].astype(jnp.int32)
    bcol = edges[:, 1].astype(jnp.int32)
    r_list, c_list, w_list = _edge_entries(a, bcol)

    deg_i = jnp.clip(degree.astype(jnp.int32).reshape(-1), 0, _NPRED)
    jj = jnp.arange(_NPRED, dtype=jnp.int32)
    mask = (jj[None, :] < deg_i[:, None]).astype(jnp.float32)

    w1p = jnp.zeros((_D, 128), jnp.float32).at[:, :_HID].set(cls_w1)
    b1p = jnp.zeros((128,), jnp.float32).at[_HID].set(1.0)
    xg1 = _matmul(feat, w1p, b1p, bm=1000)
    xg1p = jnp.concatenate([xg1, jnp.zeros((_NRACC - _N, 128), jnp.float32)])
    x1a = xg1[:, :_HID]
    x1b = _matmul(gen_feat.reshape(-1, _D), cls_w1, bm=1000)

    acc1full = _sc_spmm(r_list, c_list, w_list, xg1p)
    acc1 = acc1full[:_N, :_HID]
    rowsum = acc1full[:_N, _HID] + deg_i.astype(jnp.float32) + 1.0
    rinv = 1.0 / rowsum

    m1 = jnp.sum(mask[:, :, None] * x1b.reshape(_N, _NPRED, _HID), axis=1)
    h2a = jnp.maximum((acc1 + m1 + x1a) * rinv[:, None] + cls_b1, 0.0)
    minv = 1.0 / (1.0 + mask)
    h2b = jnp.maximum(
        (x1b.reshape(_N, _NPRED, _HID)
         + mask[:, :, None] * x1a[:, None, :]) * minv[:, :, None]
        + cls_b1, 0.0)

    w2p = jnp.zeros((_HID, 128), jnp.float32).at[:, :_NCLASS].set(cls_w2)
    xg2 = _matmul(h2a, w2p, bm=1000)
    xg2p = jnp.concatenate([xg2, jnp.zeros((_NRACC - _N, 128), jnp.float32)])
    y2a = xg2[:, :_NCLASS]
    y2b = _matmul(h2b.reshape(-1, _HID), cls_w2, bm=1000)

    acc2 = _sc_spmm(r_list, c_list, w_list, xg2p)[:_N, :_NCLASS]
    m2 = jnp.sum(mask[:, :, None] * y2b.reshape(_N, _NPRED, _NCLASS), axis=1)
    outa = jnp.maximum((acc2 + m2 + y2a) * rinv[:, None] + cls_b2, 0.0)
    outb = jnp.maximum(
        (y2b.reshape(_N, _NPRED, _NCLASS)
         + mask[:, :, None] * y2a[:, None, :]) * minv[:, :, None]
        + cls_b2, 0.0)
    nc_pred = jnp.concatenate([outa, outb.reshape(-1, _NCLASS)], axis=0)

    return (degree, gen_feat, nc_pred)
